# baseline (speedup 1.0000x reference)
import functools

import jax
import jax.numpy as jnp
from jax import lax
from jax.experimental import pallas as pl
from jax.experimental.pallas import tpu as pltpu
from jax.experimental.pallas import tpu_sc as plsc

N = 1966080
BETA = 1.0 / 9.0

NC = 2
NS = 16
L = 16
NW = NC * NS
LANES = 128
ROWS = N // LANES

R_SC = 6144
SC_CR = 32
SC_RPW = R_SC // NW
SC_CHUNKS = SC_RPW // SC_CR
assert SC_RPW * NW == R_SC
assert SC_CHUNKS * SC_CR == SC_RPW and SC_CHUNKS % 2 == 0

BR = 1024
TC_ROW0 = R_SC
TC_BLOCKS = (ROWS - TC_ROW0) // BR
assert TC_ROW0 % BR == 0 and TC_BLOCKS * BR + TC_ROW0 == ROWS

_mesh = plsc.VectorSubcoreMesh(
    core_axis_name="c", subcore_axis_name="s", num_cores=NC, num_subcores=NS
)


def _vf(c):
    return jnp.full((L,), c, jnp.float32)


def _vi(c):
    return jnp.full((L,), c, jnp.int32)


@functools.partial(
    pl.kernel,
    out_type=jax.ShapeDtypeStruct((NW * 48,), jnp.float32),
    mesh=_mesh,
    compiler_params=pltpu.CompilerParams(use_tc_tiling_on_sc=True),
    scratch_types=[
        pltpu.VMEM((2, SC_CR, LANES), jnp.float32),
        pltpu.VMEM((2, SC_CR, LANES), jnp.int32),
        pltpu.VMEM((2, 4 * SC_CR, LANES), jnp.float32),
        pltpu.VMEM((2, 4 * SC_CR, LANES), jnp.float32),
        pltpu.VMEM((48,), jnp.float32),
        pltpu.SemaphoreType.DMA,
        pltpu.SemaphoreType.DMA,
    ],
)
def _rpn_sc(obj_hbm, lab_hbm, pred_hbm, gt_hbm, out_hbm,
            obj_v, lab_v, pred_v, gt_v, part_v, sem0, sem1):
    wid = lax.axis_index("c") * NS + lax.axis_index("s")
    base = wid * SC_RPW
    sems = (sem0, sem1)

    def _copies(g, slot):
        sem = sems[slot]
        r0 = base + g * SC_CR
        return (
            pltpu.make_async_copy(obj_hbm.at[pl.ds(r0, SC_CR)], obj_v.at[slot], sem),
            pltpu.make_async_copy(lab_hbm.at[pl.ds(r0, SC_CR)], lab_v.at[slot], sem),
            pltpu.make_async_copy(pred_hbm.at[pl.ds(4 * r0, 4 * SC_CR)], pred_v.at[slot], sem),
            pltpu.make_async_copy(gt_hbm.at[pl.ds(4 * r0, 4 * SC_CR)], gt_v.at[slot], sem),
        )

    def _start(g, slot):
        for c in _copies(g, slot):
            c.start()

    def _wait(g, slot):
        for c in _copies(g, slot):
            c.wait()

    def _group(i, slot, accs):
        acc_cls, acc_loc, acc_val = accs
        r = jax.lax.shift_right_logical(i, 3)
        col = jax.lax.shift_left(jax.lax.rem(i, 8), 4)
        x = obj_v.at[slot][r, pl.ds(col, L)]
        lab = lab_v.at[slot][r, pl.ds(col, L)]
        ones, zeros = _vf(1.0), _vf(0.0)
        posf = jnp.where(lab == _vi(1), ones, zeros)
        validf = jnp.where(lab != _vi(2), ones, zeros)
        t = jnp.exp(zeros - jnp.abs(x))
        z = t / (_vf(2.0) + t)
        w = z * z
        l1p = (_vf(2.0) * z) * (
            ones + w * (_vf(1.0 / 3.0)
                        + w * (_vf(1.0 / 5.0)
                               + w * (_vf(1.0 / 7.0) + w * _vf(1.0 / 9.0))))
        )
        ce = jnp.maximum(x, zeros) - x * posf + l1p
        acc_cls = acc_cls + ce * validf
        acc_val = acc_val + validf
        dr = jax.lax.shift_left(r, 2)
        for c in range(4):
            p = pred_v.at[slot][dr + c, pl.ds(col, L)]
            g = gt_v.at[slot][dr + c, pl.ds(col, L)]
            d = p - g
            a = jnp.abs(d)
            sl1 = jnp.where(a < _vf(BETA), _vf(0.5 / BETA) * d * d, a - _vf(0.5 * BETA))
            acc_loc = acc_loc + sl1 * posf
        return acc_cls, acc_loc, acc_val

    def _compute(slot, accs):
        return lax.fori_loop(
            0, SC_CR * (LANES // L),
            lambda i, cv: _group(i, slot, cv),
            accs,
        )

    zero = jnp.zeros((L,), jnp.float32)
    _start(0, 0)

    def outer(o, accs):
        g0 = 2 * o
        _start(g0 + 1, 1)
        _wait(g0, 0)
        accs = _compute(0, accs)

        @pl.when(o < SC_CHUNKS // 2 - 1)
        def _():
            _start(g0 + 2, 0)

        _wait(g0 + 1, 1)
        return _compute(1, accs)

    acc_cls, acc_loc, acc_val = lax.fori_loop(0, SC_CHUNKS // 2, outer, (zero, zero, zero))

    part_v[pl.ds(0, L)] = acc_cls
    part_v[pl.ds(16, L)] = acc_loc
    part_v[pl.ds(32, L)] = acc_val
    pltpu.sync_copy(part_v, out_hbm.at[pl.ds(wid * 48, 48)])


def _tc_body(obj_ref, lab_ref, pred_ref, gt_ref, rmat_ref, cls_ref, loc_ref, val_ref):
    i = pl.program_id(0)
    x = obj_ref[...]
    lab = lab_ref[...]
    posf = (lab == 1).astype(jnp.float32)
    validf = (lab != 2).astype(jnp.float32)
    ce = jnp.maximum(x, 0.0) - x * posf + jnp.log(1.0 + jnp.exp(-jnp.abs(x)))
    pc = jnp.sum(ce * validf)
    pv = jnp.sum(validf)

    d = pred_ref[...] - gt_ref[...]
    a = jnp.abs(d)
    sl1 = jnp.where(a < BETA, (0.5 / BETA) * d * d, a - 0.5 * BETA)
    labh = (lab == 1).astype(jnp.bfloat16)
    labrep = jax.lax.dot(rmat_ref[...], labh, preferred_element_type=jnp.float32)
    pl_ = jnp.sum(sl1 * labrep)

    @pl.when(i == 0)
    def _():
        cls_ref[0, 0] = 0.0
        loc_ref[0, 0] = 0.0
        val_ref[0, 0] = 0.0

    cls_ref[0, 0] += pc
    loc_ref[0, 0] += pl_
    val_ref[0, 0] += pv


def kernel(pred_objectness_logits, pred_anchor_deltas, gt_anchor_deltas, gt_labels):
    obj2 = pred_objectness_logits.reshape(ROWS, LANES)
    lab2 = gt_labels.reshape(ROWS, LANES)
    pred3 = pred_anchor_deltas.reshape(ROWS, LANES, 4).swapaxes(1, 2).reshape(4 * ROWS, LANES)
    gt3 = gt_anchor_deltas.reshape(ROWS, LANES, 4).swapaxes(1, 2).reshape(4 * ROWS, LANES)
    rmat = (jnp.arange(4 * BR)[:, None] // 4 == jnp.arange(BR)[None, :]).astype(jnp.bfloat16)

    sc_parts = _rpn_sc(obj2, lab2, pred3, gt3)

    cls_tc, loc_tc, val_tc = pl.pallas_call(
        _tc_body,
        grid=(TC_BLOCKS,),
        in_specs=[
            pl.BlockSpec((BR, LANES), lambda i: (TC_ROW0 // BR + i, 0)),
            pl.BlockSpec((BR, LANES), lambda i: (TC_ROW0 // BR + i, 0)),
            pl.BlockSpec((4 * BR, LANES), lambda i: (TC_ROW0 // BR + i, 0)),
            pl.BlockSpec((4 * BR, LANES), lambda i: (TC_ROW0 // BR + i, 0)),
            pl.BlockSpec((4 * BR, BR), lambda i: (0, 0)),
        ],
        out_specs=[
            pl.BlockSpec(memory_space=pltpu.SMEM),
            pl.BlockSpec(memory_space=pltpu.SMEM),
            pl.BlockSpec(memory_space=pltpu.SMEM),
        ],
        out_shape=[
            jax.ShapeDtypeStruct((1, 1), jnp.float32),
            jax.ShapeDtypeStruct((1, 1), jnp.float32),
            jax.ShapeDtypeStruct((1, 1), jnp.float32),
        ],
    )(obj2, lab2, pred3, gt3, rmat)

    p = sc_parts.reshape(NW, 3, L)
    loss_cls = jnp.sum(p[:, 0, :]) + cls_tc[0, 0]
    loss_loc = jnp.sum(p[:, 1, :]) + loc_tc[0, 0]
    valid = jnp.sum(p[:, 2, :]) + val_tc[0, 0]
    return (loss_cls + loss_loc) / jnp.maximum(valid, 1.0)


# speedup vs baseline: 1.1824x; 1.1824x over previous
import functools

import jax
import jax.numpy as jnp
from jax import lax
from jax.experimental import pallas as pl
from jax.experimental.pallas import tpu as pltpu
from jax.experimental.pallas import tpu_sc as plsc

N = 1966080
BETA = 1.0 / 9.0

NC = 2
NS = 16
L = 16
NW = NC * NS
LANES = 128
ROWS = N // LANES

R_SC = 6144
SC_CR = 32
SC_RPW = R_SC // NW
SC_CHUNKS = SC_RPW // SC_CR
assert SC_RPW * NW == R_SC
assert SC_CHUNKS * SC_CR == SC_RPW and SC_CHUNKS % 2 == 0

BR = 512
TC_ROW0 = R_SC
TC_BLOCKS = (ROWS - TC_ROW0) // BR
assert TC_ROW0 % BR == 0 and TC_BLOCKS * BR + TC_ROW0 == ROWS

_mesh = plsc.VectorSubcoreMesh(
    core_axis_name="c", subcore_axis_name="s", num_cores=NC, num_subcores=NS
)


def _vf(c):
    return jnp.full((L,), c, jnp.float32)


def _vi(c):
    return jnp.full((L,), c, jnp.int32)


@functools.partial(
    pl.kernel,
    out_type=jax.ShapeDtypeStruct((NW, LANES), jnp.float32),
    mesh=_mesh,
    compiler_params=pltpu.CompilerParams(use_tc_tiling_on_sc=True),
    scratch_types=[
        pltpu.VMEM((2, SC_CR, LANES), jnp.float32),
        pltpu.VMEM((2, SC_CR, LANES), jnp.int32),
        pltpu.VMEM((2, 4 * SC_CR, LANES), jnp.float32),
        pltpu.VMEM((2, 4 * SC_CR, LANES), jnp.float32),
        pltpu.VMEM((LANES,), jnp.float32),
        pltpu.SemaphoreType.DMA,
        pltpu.SemaphoreType.DMA,
    ],
)
def _rpn_sc(obj_hbm, lab_hbm, pred_hbm, gt_hbm, out_hbm,
            obj_v, lab_v, pred_v, gt_v, part_v, sem0, sem1):
    wid = lax.axis_index("c") * NS + lax.axis_index("s")
    base = wid * SC_RPW
    sems = (sem0, sem1)

    def _copies(g, slot):
        sem = sems[slot]
        r0 = base + g * SC_CR
        return (
            pltpu.make_async_copy(obj_hbm.at[pl.ds(r0, SC_CR)], obj_v.at[slot], sem),
            pltpu.make_async_copy(lab_hbm.at[pl.ds(r0, SC_CR)], lab_v.at[slot], sem),
            pltpu.make_async_copy(pred_hbm.at[pl.ds(4 * r0, 4 * SC_CR)], pred_v.at[slot], sem),
            pltpu.make_async_copy(gt_hbm.at[pl.ds(4 * r0, 4 * SC_CR)], gt_v.at[slot], sem),
        )

    def _start(g, slot):
        for c in _copies(g, slot):
            c.start()

    def _wait(g, slot):
        for c in _copies(g, slot):
            c.wait()

    def _group(i, slot, accs):
        acc_cls, acc_loc, acc_val = accs
        r = jax.lax.shift_right_logical(i, 3)
        col = jax.lax.shift_left(jax.lax.rem(i, 8), 4)
        x = obj_v.at[slot][r, pl.ds(col, L)]
        lab = lab_v.at[slot][r, pl.ds(col, L)]
        ones, zeros = _vf(1.0), _vf(0.0)
        posf = jnp.where(lab == _vi(1), ones, zeros)
        validf = jnp.where(lab != _vi(2), ones, zeros)
        t = jnp.exp(zeros - jnp.abs(x))
        z = t / (_vf(2.0) + t)
        w = z * z
        l1p = (_vf(2.0) * z) * (
            ones + w * (_vf(1.0 / 3.0)
                        + w * (_vf(1.0 / 5.0)
                               + w * (_vf(1.0 / 7.0) + w * _vf(1.0 / 9.0))))
        )
        ce = jnp.maximum(x, zeros) - x * posf + l1p
        acc_cls = acc_cls + ce * validf
        acc_val = acc_val + validf
        dr = jax.lax.shift_left(r, 2)
        for c in range(4):
            p = pred_v.at[slot][dr + c, pl.ds(col, L)]
            g = gt_v.at[slot][dr + c, pl.ds(col, L)]
            d = p - g
            a = jnp.abs(d)
            sl1 = jnp.where(a < _vf(BETA), _vf(0.5 / BETA) * d * d, a - _vf(0.5 * BETA))
            acc_loc = acc_loc + sl1 * posf
        return acc_cls, acc_loc, acc_val

    def _compute(slot, accs):
        return lax.fori_loop(
            0, SC_CR * (LANES // L),
            lambda i, cv: _group(i, slot, cv),
            accs,
        )

    zero = jnp.zeros((L,), jnp.float32)
    _start(0, 0)

    def outer(o, accs):
        g0 = 2 * o
        _start(g0 + 1, 1)
        _wait(g0, 0)
        accs = _compute(0, accs)

        @pl.when(o < SC_CHUNKS // 2 - 1)
        def _():
            _start(g0 + 2, 0)

        _wait(g0 + 1, 1)
        return _compute(1, accs)

    acc_cls, acc_loc, acc_val = lax.fori_loop(0, SC_CHUNKS // 2, outer, (zero, zero, zero))

    part_v[pl.ds(0, L)] = acc_cls
    part_v[pl.ds(16, L)] = acc_loc
    part_v[pl.ds(32, L)] = acc_val
    part_v[pl.ds(48, L)] = jnp.zeros((L,), jnp.float32)
    part_v[pl.ds(64, L)] = jnp.zeros((L,), jnp.float32)
    part_v[pl.ds(80, L)] = jnp.zeros((L,), jnp.float32)
    part_v[pl.ds(96, L)] = jnp.zeros((L,), jnp.float32)
    part_v[pl.ds(112, L)] = jnp.zeros((L,), jnp.float32)
    pltpu.sync_copy(part_v, out_hbm.at[wid])


def _tc_body(obj_ref, lab_ref, pred_ref, gt_ref, cls_ref, loc_ref, val_ref, rmat_ref):
    i = pl.program_id(0)

    @pl.when(i == 0)
    def _():
        comp = lax.broadcasted_iota(jnp.int32, (4 * BR, BR), 0)
        anch = lax.broadcasted_iota(jnp.int32, (4 * BR, BR), 1)
        rmat_ref[...] = (jax.lax.shift_right_logical(comp, 2) == anch).astype(jnp.bfloat16)
    x = obj_ref[...]
    lab = lab_ref[...]
    posf = (lab == 1).astype(jnp.float32)
    validf = (lab != 2).astype(jnp.float32)
    ce = jnp.maximum(x, 0.0) - x * posf + jnp.log(1.0 + jnp.exp(-jnp.abs(x)))
    pc = jnp.sum(ce * validf)
    pv = jnp.sum(validf)

    d = pred_ref[...] - gt_ref[...]
    a = jnp.abs(d)
    sl1 = jnp.where(a < BETA, (0.5 / BETA) * d * d, a - 0.5 * BETA)
    labh = (lab == 1).astype(jnp.bfloat16)
    labrep = jax.lax.dot(rmat_ref[...], labh, preferred_element_type=jnp.float32)
    pl_ = jnp.sum(sl1 * labrep)

    @pl.when(i == 0)
    def _():
        cls_ref[0, 0] = 0.0
        loc_ref[0, 0] = 0.0
        val_ref[0, 0] = 0.0

    cls_ref[0, 0] += pc
    loc_ref[0, 0] += pl_
    val_ref[0, 0] += pv


def kernel(pred_objectness_logits, pred_anchor_deltas, gt_anchor_deltas, gt_labels):
    obj2 = pred_objectness_logits.reshape(ROWS, LANES)
    lab2 = gt_labels.reshape(ROWS, LANES)
    pred3 = pred_anchor_deltas.reshape(ROWS, LANES, 4).swapaxes(1, 2).reshape(4 * ROWS, LANES)
    gt3 = gt_anchor_deltas.reshape(ROWS, LANES, 4).swapaxes(1, 2).reshape(4 * ROWS, LANES)
    sc_parts = _rpn_sc(obj2, lab2, pred3, gt3)

    cls_tc, loc_tc, val_tc = pl.pallas_call(
        _tc_body,
        grid=(TC_BLOCKS,),
        in_specs=[
            pl.BlockSpec((BR, LANES), lambda i: (TC_ROW0 // BR + i, 0)),
            pl.BlockSpec((BR, LANES), lambda i: (TC_ROW0 // BR + i, 0)),
            pl.BlockSpec((4 * BR, LANES), lambda i: (TC_ROW0 // BR + i, 0)),
            pl.BlockSpec((4 * BR, LANES), lambda i: (TC_ROW0 // BR + i, 0)),
        ],
        scratch_shapes=[pltpu.VMEM((4 * BR, BR), jnp.bfloat16)],
        out_specs=[
            pl.BlockSpec(memory_space=pltpu.SMEM),
            pl.BlockSpec(memory_space=pltpu.SMEM),
            pl.BlockSpec(memory_space=pltpu.SMEM),
        ],
        out_shape=[
            jax.ShapeDtypeStruct((1, 1), jnp.float32),
            jax.ShapeDtypeStruct((1, 1), jnp.float32),
            jax.ShapeDtypeStruct((1, 1), jnp.float32),
        ],
    )(obj2, lab2, pred3, gt3)

    p = sc_parts.reshape(NW, 8, L)
    loss_cls = jnp.sum(p[:, 0, :]) + cls_tc[0, 0]
    loss_loc = jnp.sum(p[:, 1, :]) + loc_tc[0, 0]
    valid = jnp.sum(p[:, 2, :]) + val_tc[0, 0]
    return (loss_cls + loss_loc) / jnp.maximum(valid, 1.0)
